# Initial kernel scaffold; baseline (speedup 1.0000x reference)
import jax
import jax.numpy as jnp
from jax.experimental import pallas as pl


def kernel(x, pos, batch, W, b):
    raise NotImplementedError("write your pallas kernel here")


# speedup vs baseline: 7.3385x; 7.3385x over previous
import jax
import jax.numpy as jnp
from jax.experimental import pallas as pl
from jax.experimental.pallas import tpu as pltpu

_N = 4096
_K = 16
_D = 128
_BLK = 256
_BIG = 1e30
_TAKEN = 2e30


def _body(pos_blk_ref, pos_t_ref, pos_ref, bat_col_ref, bat_row_ref,
          x_ref, w_ref, b_ref, out_ref):
    pos_blk = pos_blk_ref[...]
    pos_t = pos_t_ref[...]
    sq_i = jnp.sum(pos_blk * pos_blk, axis=1, keepdims=True)
    sq_j = jnp.sum(pos_t * pos_t, axis=0, keepdims=True)
    dots = jax.lax.dot_general(pos_blk, pos_t, (((1,), (0,)), ((), ())),
                               preferred_element_type=jnp.float32)
    d2 = sq_i + sq_j - 2.0 * dots
    mask = bat_col_ref[...] != bat_row_ref[...]
    d2 = jnp.where(mask, _BIG, d2)

    col = jax.lax.broadcasted_iota(jnp.int32, (_BLK, _N), 1)
    pos_all = pos_ref[...]
    x_all = x_ref[...]
    w = w_ref[...]
    bb = b_ref[...]

    for k in range(_K):
        m = jnp.min(d2, axis=1, keepdims=True)
        cand = jnp.where(d2 == m, col, jnp.int32(_N))
        amin = jnp.min(cand, axis=1, keepdims=True)
        onehot_b = col == amin
        onehot = onehot_b.astype(jnp.float32)
        d2 = jnp.where(onehot_b, _TAKEN, d2)
        pos_j = jax.lax.dot_general(onehot, pos_all,
                                    (((1,), (0,)), ((), ())),
                                    preferred_element_type=jnp.float32)
        x_j = jax.lax.dot_general(onehot, x_all,
                                  (((1,), (0,)), ((), ())),
                                  preferred_element_type=jnp.float32)
        rel = pos_blk - pos_j
        dist = jnp.sqrt(jnp.sum(rel * rel, axis=1, keepdims=True) + 1e-12)
        spatial = jnp.concatenate([pos_blk, pos_j, rel, dist], axis=1)
        enc = jax.lax.dot_general(spatial, w, (((1,), (0,)), ((), ())),
                                  preferred_element_type=jnp.float32)
        enc = jnp.maximum(enc + bb, 0.0)
        out_ref[:, k, 0:_D] = enc
        out_ref[:, k, _D:2 * _D] = x_j


def kernel(x, pos, batch, W, b):
    n = pos.shape[0]
    bat = batch.astype(jnp.int32)
    bat_col = bat.reshape(n, 1)
    bat_row = bat.reshape(1, n)
    pos_t = pos.T
    b2 = b.reshape(1, _D)

    grid = (n // _BLK,)
    out = pl.pallas_call(
        _body,
        grid=grid,
        in_specs=[
            pl.BlockSpec((_BLK, 3), lambda i: (i, 0)),
            pl.BlockSpec((3, n), lambda i: (0, 0)),
            pl.BlockSpec((n, 3), lambda i: (0, 0)),
            pl.BlockSpec((_BLK, 1), lambda i: (i, 0)),
            pl.BlockSpec((1, n), lambda i: (0, 0)),
            pl.BlockSpec((n, _D), lambda i: (0, 0)),
            pl.BlockSpec((10, _D), lambda i: (0, 0)),
            pl.BlockSpec((1, _D), lambda i: (0, 0)),
        ],
        out_specs=pl.BlockSpec((_BLK, _K, 2 * _D), lambda i: (i, 0, 0)),
        out_shape=jax.ShapeDtypeStruct((n, _K, 2 * _D), jnp.float32),
    )(pos, pos_t, pos, bat_col, bat_row, x, W, b2)
    return out
